# baseline (speedup 1.0000x reference)
import functools

import jax
import jax.numpy as jnp
from jax import lax
from jax.experimental import pallas as pl
from jax.experimental.pallas import tpu as pltpu
from jax.experimental.pallas import tpu_sc as plsc

_MAXR = 128
_D = 1024
_B = 32 * 2048
_NC, _NS = 2, 16
_NW = _NC * _NS
_BPW = _B // _NW
_CHUNK = 64
_NCHUNK = _BPW // _CHUNK
_LANES = 16


def _emb_body(idx_hbm, table_hbm, out_hbm, idx_v, buf, gsem, wsem):
    wid = lax.axis_index("s") * _NC + lax.axis_index("c")
    base = wid * _BPW

    pltpu.sync_copy(idx_hbm.at[pl.ds(base, _BPW)], idx_v)

    def clamp_body(i, carry):
        sl = pl.ds(i * _LANES, _LANES)
        v = idx_v[sl]
        idx_v[sl] = jnp.minimum(jnp.maximum(v, -_MAXR), _MAXR) + _MAXR
        return carry

    lax.fori_loop(0, _BPW // _LANES, clamp_body, 0)

    def chunk_body(g, carry):
        off = g * _CHUNK
        idx_slice = idx_v.at[pl.ds(off, _CHUNK)]
        pltpu.async_copy(table_hbm.at[idx_slice], buf, gsem).wait()
        pltpu.async_copy(buf, out_hbm.at[pl.ds(base + off, _CHUNK)], wsem).wait()
        return carry

    lax.fori_loop(0, _NCHUNK, chunk_body, 0)


@jax.jit
def _emb_call(idx_flat, W):
    mesh = plsc.VectorSubcoreMesh(core_axis_name="c", subcore_axis_name="s")
    fn = functools.partial(
        pl.kernel,
        mesh=mesh,
        out_type=jax.ShapeDtypeStruct((_B, _D), jnp.float32),
        scratch_types=[
            pltpu.VMEM((_BPW,), jnp.int32),
            pltpu.VMEM((_CHUNK, _D), jnp.float32),
            pltpu.SemaphoreType.DMA,
            pltpu.SemaphoreType.DMA,
        ],
    )(_emb_body)
    return fn(idx_flat, W)


def kernel(relative_dis, W):
    idx_flat = relative_dis.reshape(-1).astype(jnp.int32)
    out = _emb_call(idx_flat, W)
    return out.reshape(relative_dis.shape + (_D,))
